# baseline (speedup 1.0000x reference)
import functools

import jax
import jax.numpy as jnp
from jax import lax
from jax.experimental import pallas as pl
from jax.experimental.pallas import tpu as pltpu
from jax.experimental.pallas import tpu_sc as plsc

N = 10000
D_IN = 256
D_H = 512
D_OUT = 256

NC, NS = 2, 16
HALF = 128
CHUNK = 128
N_CHUNKS = 81
DEPTH = 3
EPT = CHUNK * N_CHUNKS
E_PAD = EPT * NS
ACC_N = 10016
ZROWS = 640
Z_LAST = ACC_N - 15 * ZROWS
WB_LAST = N - 15 * ZROWS

_mesh = plsc.VectorSubcoreMesh(core_axis_name="c", subcore_axis_name="s",
                               num_cores=NC, num_subcores=NS)


def _agg_body(with_cnt, x0, x1, src, dst, zrows, zcnt, *rest):
    if with_cnt:
        (agg0, agg1, cnt_out, *rest) = rest
        srcs, rest = rest[:DEPTH], rest[DEPTH:]
        dsts, rest = rest[:DEPTH], rest[DEPTH:]
        rows, rest = rest[:DEPTH], rest[DEPTH:]
        ones_v, acc_sh, cnt_sh, *sems = rest
    else:
        agg0, agg1, *rest = rest
        srcs, rest = rest[:DEPTH], rest[DEPTH:]
        dsts, rest = rest[:DEPTH], rest[DEPTH:]
        rows, rest = rest[:DEPTH], rest[DEPTH:]
        acc_sh, *sems = rest
    semi = sems[:DEPTH]
    semg = sems[DEPTH:]
    c = lax.axis_index("c")
    s = lax.axis_index("s")

    @pl.when(s < NS - 1)
    def _():
        pltpu.sync_copy(zrows, acc_sh.at[pl.ds(s * ZROWS, ZROWS)])

    @pl.when(s == NS - 1)
    def _():
        pltpu.sync_copy(zrows.at[pl.ds(0, Z_LAST)],
                        acc_sh.at[pl.ds((NS - 1) * ZROWS, Z_LAST)])
    if with_cnt:
        @pl.when(s == 0)
        def _():
            pltpu.sync_copy(zcnt, cnt_sh)
        for j in range(CHUNK // 16):
            ones_v[pl.ds(j * 16, 16)] = jnp.ones((16,), jnp.float32)
    plsc.subcore_barrier()

    def edge_loop(x_half):
        def idx_load(i, b):
            pltpu.async_copy(src.at[s, i], srcs[b], semi[b])
            pltpu.async_copy(dst.at[s, i], dsts[b], semi[b])

        def idx_wait(b):
            pltpu.make_async_copy(src.at[s, 0], srcs[b], semi[b]).wait()
            pltpu.make_async_copy(dst.at[s, 0], dsts[b], semi[b]).wait()

        def gather(b):
            pltpu.async_copy(x_half.at[srcs[b]], rows[b], semg[b])

        def gather_wait(b):
            pltpu.make_async_copy(x_half.at[srcs[0]], rows[b], semg[b]).wait()

        def scatter(b):
            pltpu.sync_copy(rows[b], acc_sh.at[dsts[b]], add=True)
            if with_cnt:
                pltpu.sync_copy(ones_v, cnt_sh.at[dsts[b]], add=True)

        for k in range(DEPTH - 1):
            idx_load(k, k)
        for k in range(DEPTH - 1):
            idx_wait(k)
            gather(k)
        idx_load(DEPTH - 1, DEPTH - 1)

        def step(j, carry):
            i0 = DEPTH * j
            for k in range(DEPTH):
                i = i0 + k
                b = k
                bp = (k - 1) % DEPTH

                @pl.when(i + DEPTH - 1 < N_CHUNKS)
                def _(b=bp):
                    idx_wait(b)
                    gather(b)

                gather_wait(b)
                scatter(b)

                @pl.when(i + DEPTH < N_CHUNKS)
                def _(i=i, b=b):
                    idx_load(i + DEPTH, b)
            return carry
        lax.fori_loop(0, N_CHUNKS // DEPTH, step, 0)

    @pl.when(c == 0)
    def _():
        edge_loop(x0)

    @pl.when(c == 1)
    def _():
        edge_loop(x1)

    plsc.subcore_barrier()

    agg_out = [agg0, agg1]
    for ci in range(NC):
        @pl.when((c == ci) & (s < NS - 1))
        def _(ci=ci):
            wb = pl.ds(s * ZROWS, ZROWS)
            pltpu.sync_copy(acc_sh.at[wb], agg_out[ci].at[wb])

        @pl.when((c == ci) & (s == NS - 1))
        def _(ci=ci):
            wb = pl.ds((NS - 1) * ZROWS, WB_LAST)
            pltpu.sync_copy(acc_sh.at[wb], agg_out[ci].at[wb])

    if with_cnt:
        @pl.when((c == 0) & (s == 0))
        def _():
            pltpu.sync_copy(cnt_sh, cnt_out)


def _make_agg(with_cnt):
    out_type = [jax.ShapeDtypeStruct((N, HALF), jnp.float32),
                jax.ShapeDtypeStruct((N, HALF), jnp.float32)]
    scratch = ([pltpu.VMEM((CHUNK,), jnp.int32)] * (2 * DEPTH)
               + [pltpu.VMEM((CHUNK, HALF), jnp.float32)] * DEPTH)
    if with_cnt:
        out_type = out_type + [jax.ShapeDtypeStruct((ACC_N,), jnp.float32)]
        scratch = scratch + [pltpu.VMEM((CHUNK,), jnp.float32),
                             pltpu.VMEM_SHARED((ACC_N, HALF), jnp.float32),
                             pltpu.VMEM_SHARED((ACC_N,), jnp.float32)]
    else:
        scratch = scratch + [pltpu.VMEM_SHARED((ACC_N, HALF), jnp.float32)]
    scratch = scratch + [pltpu.SemaphoreType.DMA] * (2 * DEPTH)
    return pl.kernel(functools.partial(_agg_body, with_cnt),
                     out_type=out_type, mesh=_mesh, scratch_types=scratch,
                     name="sc_segment_mean" + ("_cnt" if with_cnt else ""))


_agg_cnt = _make_agg(True)
_agg = _make_agg(False)


_R = 400


def _l1_body(x_ref, a0_ref, a1_ref, cnt_ref, w1l_ref, b1_ref, w1r_ref,
             w2l_ref, h_ref, hw0_ref, hw1_ref):
    r = 1.0 / jnp.maximum(cnt_ref[...], 1.0)
    a = jnp.concatenate([a0_ref[...], a1_ref[...]], axis=1) * r
    h = (jnp.dot(a, w1l_ref[...], preferred_element_type=jnp.float32)
         + b1_ref[...]
         + jnp.dot(x_ref[...], w1r_ref[...], preferred_element_type=jnp.float32))
    h = jnp.maximum(h, 0.0)
    h_ref[...] = h
    hw = jnp.dot(h, w2l_ref[...], preferred_element_type=jnp.float32)
    hw0_ref[...] = hw[:, :HALF]
    hw1_ref[...] = hw[:, HALF:]


def _l1(x, a0, a1, cnt2d, W1l, b1_2d, W1r, W2l):
    grid = (N // _R,)
    return pl.pallas_call(
        _l1_body,
        grid=grid,
        in_specs=[
            pl.BlockSpec((_R, D_IN), lambda i: (i, 0)),
            pl.BlockSpec((_R, HALF), lambda i: (i, 0)),
            pl.BlockSpec((_R, HALF), lambda i: (i, 0)),
            pl.BlockSpec((_R, 1), lambda i: (i, 0)),
            pl.BlockSpec((D_IN, D_H), lambda i: (0, 0)),
            pl.BlockSpec((1, D_H), lambda i: (0, 0)),
            pl.BlockSpec((D_IN, D_H), lambda i: (0, 0)),
            pl.BlockSpec((D_H, D_OUT), lambda i: (0, 0)),
        ],
        out_specs=[
            pl.BlockSpec((_R, D_H), lambda i: (i, 0)),
            pl.BlockSpec((_R, HALF), lambda i: (i, 0)),
            pl.BlockSpec((_R, HALF), lambda i: (i, 0)),
        ],
        out_shape=[
            jax.ShapeDtypeStruct((N, D_H), jnp.float32),
            jax.ShapeDtypeStruct((N, HALF), jnp.float32),
            jax.ShapeDtypeStruct((N, HALF), jnp.float32),
        ],
        name="tc_layer1",
    )(x, a0, a1, cnt2d, W1l, b1_2d, W1r, W2l)


def _l2_body(h_ref, a0_ref, a1_ref, cnt_ref, w2r_ref, b2_ref,
             emb_ref, pooled_ref):
    i = pl.program_id(0)
    r = 1.0 / jnp.maximum(cnt_ref[...], 1.0)
    a = jnp.concatenate([a0_ref[...], a1_ref[...]], axis=1) * r
    emb = (a + b2_ref[...]
           + jnp.dot(h_ref[...], w2r_ref[...], preferred_element_type=jnp.float32))
    emb_ref[...] = emb

    @pl.when(i == 0)
    def _():
        pooled_ref[...] = jnp.zeros_like(pooled_ref)

    pooled_ref[...] += jnp.sum(emb, axis=0, keepdims=True) * (1.0 / N)


def _l2(h, a0, a1, cnt2d, W2r, b2_2d):
    grid = (N // _R,)
    return pl.pallas_call(
        _l2_body,
        grid=grid,
        in_specs=[
            pl.BlockSpec((_R, D_H), lambda i: (i, 0)),
            pl.BlockSpec((_R, HALF), lambda i: (i, 0)),
            pl.BlockSpec((_R, HALF), lambda i: (i, 0)),
            pl.BlockSpec((_R, 1), lambda i: (i, 0)),
            pl.BlockSpec((D_H, D_OUT), lambda i: (0, 0)),
            pl.BlockSpec((1, D_OUT), lambda i: (0, 0)),
        ],
        out_specs=[
            pl.BlockSpec((_R, D_OUT), lambda i: (i, 0)),
            pl.BlockSpec((1, D_OUT), lambda i: (0, 0)),
        ],
        out_shape=[
            jax.ShapeDtypeStruct((N, D_OUT), jnp.float32),
            jax.ShapeDtypeStruct((1, D_OUT), jnp.float32),
        ],
        name="tc_layer2",
    )(h, a0, a1, cnt2d, W2r, b2_2d)


def kernel(x, edge_index, W1l, b1, W1r, W2l, b2, W2r):
    E = edge_index.shape[1]
    pad = E_PAD - E
    src = jnp.concatenate([edge_index[0], jnp.zeros((pad,), jnp.int32)])
    pad_dst = N + (jnp.arange(pad, dtype=jnp.int32) % (ACC_N - N))
    dst = jnp.concatenate([edge_index[1], pad_dst])
    src = src.reshape(NS, N_CHUNKS, CHUNK)
    dst = dst.reshape(NS, N_CHUNKS, CHUNK)
    x0 = x[:, :HALF]
    x1 = x[:, HALF:]
    zrows = jnp.zeros((ZROWS, HALF), jnp.float32)
    zcnt = jnp.zeros((ACC_N,), jnp.float32)

    agg0, agg1, cnt = _agg_cnt(x0, x1, src, dst, zrows, zcnt)
    cnt2d = cnt[:N].reshape(N, 1)
    h, hw0, hw1 = _l1(x, agg0, agg1, cnt2d, W1l, b1.reshape(1, D_H), W1r, W2l)
    g0, g1 = _agg(hw0, hw1, src, dst, zrows, zcnt)
    emb, pooled = _l2(h, g0, g1, cnt2d, W2r, b2.reshape(1, D_OUT))
    return (pooled, emb)


# speedup vs baseline: 1.1966x; 1.1966x over previous
import functools

import jax
import jax.numpy as jnp
from jax import lax
from jax.experimental import pallas as pl
from jax.experimental.pallas import tpu as pltpu
from jax.experimental.pallas import tpu_sc as plsc

N = 10000
D_IN = 256
D_H = 512
D_OUT = 256

NC, NS = 2, 16
HALF = 128
CHUNK = 80
N_CHUNKS = 128
DEPTH = 4
EPT = CHUNK * N_CHUNKS
E_PAD = EPT * NS
ACC_N = 10240
ZROWS = 640
Z_LAST = ACC_N - 15 * ZROWS
WB_LAST = N - 15 * ZROWS

_mesh = plsc.VectorSubcoreMesh(core_axis_name="c", subcore_axis_name="s",
                               num_cores=NC, num_subcores=NS)


def _agg_body(with_cnt, x0, x1, src, dst, zrows, zcnt, *rest):
    if with_cnt:
        (agg0, agg1, cnt_out, *rest) = rest
        srcs, rest = rest[:DEPTH], rest[DEPTH:]
        dsts, rest = rest[:DEPTH], rest[DEPTH:]
        rows, rest = rest[:DEPTH], rest[DEPTH:]
        ones_v, acc_sh, cnt_sh, *sems = rest
    else:
        agg0, agg1, *rest = rest
        srcs, rest = rest[:DEPTH], rest[DEPTH:]
        dsts, rest = rest[:DEPTH], rest[DEPTH:]
        rows, rest = rest[:DEPTH], rest[DEPTH:]
        acc_sh, *sems = rest
    semi = sems[:DEPTH]
    semg = sems[DEPTH:]
    c = lax.axis_index("c")
    s = lax.axis_index("s")

    @pl.when(s < NS - 1)
    def _():
        pltpu.sync_copy(zrows, acc_sh.at[pl.ds(s * ZROWS, ZROWS)])

    @pl.when(s == NS - 1)
    def _():
        pltpu.sync_copy(zrows.at[pl.ds(0, Z_LAST)],
                        acc_sh.at[pl.ds((NS - 1) * ZROWS, Z_LAST)])
    if with_cnt:
        @pl.when(s == 0)
        def _():
            pltpu.sync_copy(zcnt, cnt_sh)
        for j in range(CHUNK // 16):
            ones_v[pl.ds(j * 16, 16)] = jnp.ones((16,), jnp.float32)
    plsc.subcore_barrier()

    def edge_loop(x_half):
        def idx_load(i, b):
            pltpu.async_copy(src.at[s, i], srcs[b], semi[b])
            pltpu.async_copy(dst.at[s, i], dsts[b], semi[b])

        def idx_wait(b):
            pltpu.make_async_copy(src.at[s, 0], srcs[b], semi[b]).wait()
            pltpu.make_async_copy(dst.at[s, 0], dsts[b], semi[b]).wait()

        def gather(b):
            pltpu.async_copy(x_half.at[srcs[b]], rows[b], semg[b])

        def gather_wait(b):
            pltpu.make_async_copy(x_half.at[srcs[0]], rows[b], semg[b]).wait()

        def scatter(b):
            pltpu.sync_copy(rows[b], acc_sh.at[dsts[b]], add=True)
            if with_cnt:
                pltpu.sync_copy(ones_v, cnt_sh.at[dsts[b]], add=True)

        for k in range(DEPTH - 1):
            idx_load(k, k)
        for k in range(DEPTH - 1):
            idx_wait(k)
            gather(k)
        idx_load(DEPTH - 1, DEPTH - 1)

        def step(j, carry):
            i0 = DEPTH * j
            for k in range(DEPTH):
                i = i0 + k
                b = k
                bp = (k - 1) % DEPTH

                @pl.when(i + DEPTH - 1 < N_CHUNKS)
                def _(b=bp):
                    idx_wait(b)
                    gather(b)

                gather_wait(b)
                scatter(b)

                @pl.when(i + DEPTH < N_CHUNKS)
                def _(i=i, b=b):
                    idx_load(i + DEPTH, b)
            return carry
        lax.fori_loop(0, N_CHUNKS // DEPTH, step, 0)

    @pl.when(c == 0)
    def _():
        edge_loop(x0)

    @pl.when(c == 1)
    def _():
        edge_loop(x1)

    plsc.subcore_barrier()

    agg_out = [agg0, agg1]
    for ci in range(NC):
        @pl.when((c == ci) & (s < NS - 1))
        def _(ci=ci):
            wb = pl.ds(s * ZROWS, ZROWS)
            pltpu.sync_copy(acc_sh.at[wb], agg_out[ci].at[wb])

        @pl.when((c == ci) & (s == NS - 1))
        def _(ci=ci):
            wb = pl.ds((NS - 1) * ZROWS, WB_LAST)
            pltpu.sync_copy(acc_sh.at[wb], agg_out[ci].at[wb])

    if with_cnt:
        @pl.when((c == 0) & (s == 0))
        def _():
            pltpu.sync_copy(cnt_sh, cnt_out)


def _make_agg(with_cnt):
    out_type = [jax.ShapeDtypeStruct((N, HALF), jnp.float32),
                jax.ShapeDtypeStruct((N, HALF), jnp.float32)]
    scratch = ([pltpu.VMEM((CHUNK,), jnp.int32)] * (2 * DEPTH)
               + [pltpu.VMEM((CHUNK, HALF), jnp.float32)] * DEPTH)
    if with_cnt:
        out_type = out_type + [jax.ShapeDtypeStruct((ACC_N,), jnp.float32)]
        scratch = scratch + [pltpu.VMEM((CHUNK,), jnp.float32),
                             pltpu.VMEM_SHARED((ACC_N, HALF), jnp.float32),
                             pltpu.VMEM_SHARED((ACC_N,), jnp.float32)]
    else:
        scratch = scratch + [pltpu.VMEM_SHARED((ACC_N, HALF), jnp.float32)]
    scratch = scratch + [pltpu.SemaphoreType.DMA] * (2 * DEPTH)
    return pl.kernel(functools.partial(_agg_body, with_cnt),
                     out_type=out_type, mesh=_mesh, scratch_types=scratch,
                     name="sc_segment_mean" + ("_cnt" if with_cnt else ""))


_agg_cnt = _make_agg(True)
_agg = _make_agg(False)


_R = 400


def _l1_body(x_ref, a0_ref, a1_ref, cnt_ref, w1l_ref, b1_ref, w1r_ref,
             w2l_ref, h_ref, hw0_ref, hw1_ref):
    r = 1.0 / jnp.maximum(cnt_ref[...], 1.0)
    a = jnp.concatenate([a0_ref[...], a1_ref[...]], axis=1) * r
    h = (jnp.dot(a, w1l_ref[...], preferred_element_type=jnp.float32)
         + b1_ref[...]
         + jnp.dot(x_ref[...], w1r_ref[...], preferred_element_type=jnp.float32))
    h = jnp.maximum(h, 0.0)
    h_ref[...] = h
    hw = jnp.dot(h, w2l_ref[...], preferred_element_type=jnp.float32)
    hw0_ref[...] = hw[:, :HALF]
    hw1_ref[...] = hw[:, HALF:]


def _l1(x, a0, a1, cnt2d, W1l, b1_2d, W1r, W2l):
    grid = (N // _R,)
    return pl.pallas_call(
        _l1_body,
        grid=grid,
        in_specs=[
            pl.BlockSpec((_R, D_IN), lambda i: (i, 0)),
            pl.BlockSpec((_R, HALF), lambda i: (i, 0)),
            pl.BlockSpec((_R, HALF), lambda i: (i, 0)),
            pl.BlockSpec((_R, 1), lambda i: (i, 0)),
            pl.BlockSpec((D_IN, D_H), lambda i: (0, 0)),
            pl.BlockSpec((1, D_H), lambda i: (0, 0)),
            pl.BlockSpec((D_IN, D_H), lambda i: (0, 0)),
            pl.BlockSpec((D_H, D_OUT), lambda i: (0, 0)),
        ],
        out_specs=[
            pl.BlockSpec((_R, D_H), lambda i: (i, 0)),
            pl.BlockSpec((_R, HALF), lambda i: (i, 0)),
            pl.BlockSpec((_R, HALF), lambda i: (i, 0)),
        ],
        out_shape=[
            jax.ShapeDtypeStruct((N, D_H), jnp.float32),
            jax.ShapeDtypeStruct((N, HALF), jnp.float32),
            jax.ShapeDtypeStruct((N, HALF), jnp.float32),
        ],
        name="tc_layer1",
    )(x, a0, a1, cnt2d, W1l, b1_2d, W1r, W2l)


def _l2_body(h_ref, a0_ref, a1_ref, cnt_ref, w2r_ref, b2_ref,
             emb_ref, pooled_ref):
    i = pl.program_id(0)
    r = 1.0 / jnp.maximum(cnt_ref[...], 1.0)
    a = jnp.concatenate([a0_ref[...], a1_ref[...]], axis=1) * r
    emb = (a + b2_ref[...]
           + jnp.dot(h_ref[...], w2r_ref[...], preferred_element_type=jnp.float32))
    emb_ref[...] = emb

    @pl.when(i == 0)
    def _():
        pooled_ref[...] = jnp.zeros_like(pooled_ref)

    pooled_ref[...] += jnp.sum(emb, axis=0, keepdims=True) * (1.0 / N)


def _l2(h, a0, a1, cnt2d, W2r, b2_2d):
    grid = (N // _R,)
    return pl.pallas_call(
        _l2_body,
        grid=grid,
        in_specs=[
            pl.BlockSpec((_R, D_H), lambda i: (i, 0)),
            pl.BlockSpec((_R, HALF), lambda i: (i, 0)),
            pl.BlockSpec((_R, HALF), lambda i: (i, 0)),
            pl.BlockSpec((_R, 1), lambda i: (i, 0)),
            pl.BlockSpec((D_H, D_OUT), lambda i: (0, 0)),
            pl.BlockSpec((1, D_OUT), lambda i: (0, 0)),
        ],
        out_specs=[
            pl.BlockSpec((_R, D_OUT), lambda i: (i, 0)),
            pl.BlockSpec((1, D_OUT), lambda i: (0, 0)),
        ],
        out_shape=[
            jax.ShapeDtypeStruct((N, D_OUT), jnp.float32),
            jax.ShapeDtypeStruct((1, D_OUT), jnp.float32),
        ],
        name="tc_layer2",
    )(h, a0, a1, cnt2d, W2r, b2_2d)


def kernel(x, edge_index, W1l, b1, W1r, W2l, b2, W2r):
    E = edge_index.shape[1]
    pad = E_PAD - E
    src = jnp.concatenate([edge_index[0], jnp.zeros((pad,), jnp.int32)])
    pad_dst = N + (jnp.arange(pad, dtype=jnp.int32) % (ACC_N - N))
    dst = jnp.concatenate([edge_index[1], pad_dst])
    src = src.reshape(NS, N_CHUNKS, CHUNK)
    dst = dst.reshape(NS, N_CHUNKS, CHUNK)
    x0 = x[:, :HALF]
    x1 = x[:, HALF:]
    zrows = jnp.zeros((ZROWS, HALF), jnp.float32)
    zcnt = jnp.zeros((ACC_N,), jnp.float32)

    agg0, agg1, cnt = _agg_cnt(x0, x1, src, dst, zrows, zcnt)
    cnt2d = cnt[:N].reshape(N, 1)
    h, hw0, hw1 = _l1(x, agg0, agg1, cnt2d, W1l, b1.reshape(1, D_H), W1r, W2l)
    g0, g1 = _agg(hw0, hw1, src, dst, zrows, zcnt)
    emb, pooled = _l2(h, g0, g1, cnt2d, W2r, b2.reshape(1, D_OUT))
    return (pooled, emb)
